# Initial kernel scaffold; baseline (speedup 1.0000x reference)
import jax
import jax.numpy as jnp
from jax.experimental import pallas as pl


def kernel(x, W_enc, b_enc, W_dec, b_dec):
    raise NotImplementedError("write your pallas kernel here")


# speedup vs baseline: 1.0547x; 1.0547x over previous
import jax
import jax.numpy as jnp
from jax.experimental import pallas as pl
from jax.experimental.pallas import tpu as pltpu

B = 64
D = 2048
S = 65536
K = 64

ENC_BLK = 1024
DEC_BLK = 1024
ROW_BLK = 8


def _encode_block(x_ref, bdec_ref, w_ref, benc_ref, z_ref):
    xc = x_ref[...] - bdec_ref[...]
    z_ref[...] = (
        jnp.dot(xc, w_ref[...], preferred_element_type=jnp.float32)
        + benc_ref[...]
    )


def _topk_block(z_ref, tv_ref, ti_ref, f_ref, vals_ref):
    zr = jnp.maximum(z_ref[...], 0.0)
    vals_ref[...] = zr
    col = jax.lax.broadcasted_iota(jnp.int32, (ROW_BLK, S), 1)
    kcol = jax.lax.broadcasted_iota(jnp.int32, (ROW_BLK, K), 1)

    def body(k, carry):
        tv, ti = carry
        v = vals_ref[...]
        m = jnp.max(v, axis=1, keepdims=True)
        cand = jnp.where(v == m, col, jnp.int32(S))
        amin = jnp.min(cand, axis=1, keepdims=True)
        sel = kcol == k
        tv = jnp.where(sel, m, tv)
        ti = jnp.where(sel, amin, ti)
        vals_ref[...] = jnp.where(cand == amin, -1.0, v)
        return (tv, ti)

    tv, ti = jax.lax.fori_loop(
        0, K, body,
        (jnp.zeros((ROW_BLK, K), jnp.float32),
         jnp.zeros((ROW_BLK, K), jnp.int32)),
    )
    tv_ref[...] = tv
    ti_ref[...] = ti
    f_ref[...] = jnp.where(vals_ref[...] < 0.0, zr, 0.0)


def _decode_block(f_ref, w_ref, bdec_ref, o_ref):
    @pl.when(pl.program_id(0) == 0)
    def _():
        o_ref[...] = jnp.broadcast_to(bdec_ref[...], o_ref.shape)

    o_ref[...] += jnp.dot(f_ref[...], w_ref[...],
                          preferred_element_type=jnp.float32)


def kernel(x, W_enc, b_enc, W_dec, b_dec):
    bdec2 = b_dec.reshape(1, D)
    benc2 = b_enc.reshape(1, S)

    z = pl.pallas_call(
        _encode_block,
        grid=(S // ENC_BLK,),
        in_specs=[
            pl.BlockSpec((B, D), lambda j: (0, 0)),
            pl.BlockSpec((1, D), lambda j: (0, 0)),
            pl.BlockSpec((D, ENC_BLK), lambda j: (0, j)),
            pl.BlockSpec((1, ENC_BLK), lambda j: (0, j)),
        ],
        out_specs=pl.BlockSpec((B, ENC_BLK), lambda j: (0, j)),
        out_shape=jax.ShapeDtypeStruct((B, S), jnp.float32),
    )(x, bdec2, W_enc, benc2)

    tv, ti, f = pl.pallas_call(
        _topk_block,
        grid=(B // ROW_BLK,),
        in_specs=[pl.BlockSpec((ROW_BLK, S), lambda i: (i, 0))],
        out_specs=[
            pl.BlockSpec((ROW_BLK, K), lambda i: (i, 0)),
            pl.BlockSpec((ROW_BLK, K), lambda i: (i, 0)),
            pl.BlockSpec((ROW_BLK, S), lambda i: (i, 0)),
        ],
        out_shape=[
            jax.ShapeDtypeStruct((B, K), jnp.float32),
            jax.ShapeDtypeStruct((B, K), jnp.int32),
            jax.ShapeDtypeStruct((B, S), jnp.float32),
        ],
        scratch_shapes=[pltpu.VMEM((ROW_BLK, S), jnp.float32)],
    )(z)

    x_hat = pl.pallas_call(
        _decode_block,
        grid=(S // DEC_BLK,),
        in_specs=[
            pl.BlockSpec((B, DEC_BLK), lambda j: (0, j)),
            pl.BlockSpec((DEC_BLK, D), lambda j: (j, 0)),
            pl.BlockSpec((1, D), lambda j: (0, 0)),
        ],
        out_specs=pl.BlockSpec((B, D), lambda j: (0, 0)),
        out_shape=jax.ShapeDtypeStruct((B, D), jnp.float32),
    )(f, W_dec, bdec2)

    return (x_hat, f, ti, z)


# speedup vs baseline: 1.1538x; 1.0940x over previous
import functools

import jax
import jax.numpy as jnp
from jax.experimental import pallas as pl
from jax.experimental.pallas import tpu as pltpu
from jax.experimental.pallas import tpu_sc as plsc

B = 64
D = 2048
S = 65536
K = 64

ENC_BLK = 1024
ROW_BLK = 8

NC = 2
NS = 16
NW = NC * NS
RPW = B // NW
DCH = 16
NCH = K // DCH
DV = D // 16


def _encode_block(x_ref, bdec_ref, w_ref, benc_ref, z_ref):
    xc = x_ref[...] - bdec_ref[...]
    z_ref[...] = (
        jnp.dot(xc, w_ref[...], preferred_element_type=jnp.float32)
        + benc_ref[...]
    )


def _topk_block(z_ref, tv_ref, ti_ref, f_ref, vals_ref):
    zr = jnp.maximum(z_ref[...], 0.0)
    vals_ref[...] = zr
    col = jax.lax.broadcasted_iota(jnp.int32, (ROW_BLK, S), 1)
    kcol = jax.lax.broadcasted_iota(jnp.int32, (ROW_BLK, K), 1)

    def body(k, carry):
        tv, ti = carry
        v = vals_ref[...]
        m = jnp.max(v, axis=1, keepdims=True)
        cand = jnp.where(v == m, col, jnp.int32(S))
        amin = jnp.min(cand, axis=1, keepdims=True)
        sel = kcol == k
        tv = jnp.where(sel, m, tv)
        ti = jnp.where(sel, amin, ti)
        vals_ref[...] = jnp.where(cand == amin, -1.0, v)
        return (tv, ti)

    tv, ti = jax.lax.fori_loop(
        0, K, body,
        (jnp.zeros((ROW_BLK, K), jnp.float32),
         jnp.zeros((ROW_BLK, K), jnp.int32)),
    )
    tv_ref[...] = tv
    ti_ref[...] = ti
    f_ref[...] = jnp.where(vals_ref[...] < 0.0, zr, 0.0)


def _sc_decode_body(ti_hbm, tv_hbm, wdec_hbm, bdec_hbm, out_hbm,
                    idx_v, tvv_v, rows_v, acc_v, sem):
    wid = jax.lax.axis_index("s") * NC + jax.lax.axis_index("c")
    for rr in range(RPW):
        r = wid * RPW + rr
        pltpu.sync_copy(bdec_hbm, acc_v)
        pltpu.sync_copy(tv_hbm.at[r], tvv_v)
        for c in range(NCH):
            pltpu.sync_copy(ti_hbm.at[r, c], idx_v)
            pltpu.async_copy(wdec_hbm.at[idx_v], rows_v, sem).wait()
            ws = [tvv_v[c * DCH + j, :] for j in range(DCH)]

            def dbody(d, _):
                sl = pl.ds(d * 16, 16)
                av = acc_v[sl]
                for j in range(DCH):
                    av = av + ws[j] * rows_v[j, sl]
                acc_v[sl] = av
                return 0

            jax.lax.fori_loop(0, DV, dbody, 0)
        pltpu.sync_copy(acc_v, out_hbm.at[r])


_sc_decode = functools.partial(
    pl.kernel,
    mesh=plsc.VectorSubcoreMesh(core_axis_name="c", subcore_axis_name="s"),
    out_type=jax.ShapeDtypeStruct((B, D), jnp.float32),
    scratch_types=[
        pltpu.VMEM((DCH,), jnp.int32),
        pltpu.VMEM((K, 16), jnp.float32),
        pltpu.VMEM((DCH, D), jnp.float32),
        pltpu.VMEM((D,), jnp.float32),
        pltpu.SemaphoreType.DMA,
    ],
)(_sc_decode_body)


def kernel(x, W_enc, b_enc, W_dec, b_dec):
    bdec2 = b_dec.reshape(1, D)
    benc2 = b_enc.reshape(1, S)

    z = pl.pallas_call(
        _encode_block,
        grid=(S // ENC_BLK,),
        in_specs=[
            pl.BlockSpec((B, D), lambda j: (0, 0)),
            pl.BlockSpec((1, D), lambda j: (0, 0)),
            pl.BlockSpec((D, ENC_BLK), lambda j: (0, j)),
            pl.BlockSpec((1, ENC_BLK), lambda j: (0, j)),
        ],
        out_specs=pl.BlockSpec((B, ENC_BLK), lambda j: (0, j)),
        out_shape=jax.ShapeDtypeStruct((B, S), jnp.float32),
    )(x, bdec2, W_enc, benc2)

    tv, ti, f = pl.pallas_call(
        _topk_block,
        grid=(B // ROW_BLK,),
        in_specs=[pl.BlockSpec((ROW_BLK, S), lambda i: (i, 0))],
        out_specs=[
            pl.BlockSpec((ROW_BLK, K), lambda i: (i, 0)),
            pl.BlockSpec((ROW_BLK, K), lambda i: (i, 0)),
            pl.BlockSpec((ROW_BLK, S), lambda i: (i, 0)),
        ],
        out_shape=[
            jax.ShapeDtypeStruct((B, K), jnp.float32),
            jax.ShapeDtypeStruct((B, K), jnp.int32),
            jax.ShapeDtypeStruct((B, S), jnp.float32),
        ],
        scratch_shapes=[pltpu.VMEM((ROW_BLK, S), jnp.float32)],
    )(z)

    tvx = jnp.broadcast_to(tv[:, :, None], (B, K, 16))
    x_hat = _sc_decode(ti.reshape(B, NCH, DCH), tvx, W_dec, b_dec)

    return (x_hat, f, ti, z)


# speedup vs baseline: 2.1497x; 1.8631x over previous
import functools

import jax
import jax.numpy as jnp
from jax.experimental import pallas as pl
from jax.experimental.pallas import tpu as pltpu
from jax.experimental.pallas import tpu_sc as plsc

B = 64
D = 2048
S = 65536
K = 64

ENC_BLK = 1024
ROW_BLK = 8

NC = 2
NS = 16
NW = NC * NS
RPW = B // NW
DCH = 16
NCH = K // DCH
DV = D // 16


def _encode_block(x_ref, bdec_ref, w_ref, benc_ref, z_ref):
    xc = x_ref[...] - bdec_ref[...]
    z_ref[...] = (
        jnp.dot(xc, w_ref[...], preferred_element_type=jnp.float32)
        + benc_ref[...]
    )


NSEG = 512
SEGK = 8
NCAND = NSEG * SEGK


def _topk_block(z_ref, tv_ref, ti_ref, f_ref):
    zr = jnp.maximum(z_ref[...], 0.0)
    z3 = zr.reshape(ROW_BLK, NSEG, 128)
    col3 = (jax.lax.broadcasted_iota(jnp.int32, (ROW_BLK, NSEG, 128), 1)
            * 128
            + jax.lax.broadcasted_iota(jnp.int32, (ROW_BLK, NSEG, 128), 2))

    work = z3
    cvs, cis = [], []
    for _ in range(SEGK):
        m = jnp.max(work, axis=2, keepdims=True)
        cand = jnp.where(work == m, col3, jnp.int32(S))
        am = jnp.min(cand, axis=2, keepdims=True)
        cvs.append(jnp.squeeze(m, 2))
        cis.append(jnp.squeeze(am, 2))
        work = jnp.where(cand == am, -1.0, work)

    candV = jnp.concatenate(cvs, axis=1)
    candI = jnp.concatenate(cis, axis=1)
    kcol = jax.lax.broadcasted_iota(jnp.int32, (ROW_BLK, K), 1)

    def body(k, carry):
        cv, tv, ti = carry
        m = jnp.max(cv, axis=1, keepdims=True)
        cand = jnp.where(cv == m, candI, jnp.int32(S))
        amin = jnp.min(cand, axis=1, keepdims=True)
        sel = kcol == k
        tv = jnp.where(sel, m, tv)
        ti = jnp.where(sel, amin, ti)
        cv = jnp.where(candI == amin, -1.0, cv)
        return (cv, tv, ti)

    _, tv, ti = jax.lax.fori_loop(
        0, K, body,
        (candV,
         jnp.zeros((ROW_BLK, K), jnp.float32),
         jnp.zeros((ROW_BLK, K), jnp.int32)),
    )
    tv_ref[...] = tv
    ti_ref[...] = ti
    thr = tv[:, K - 1:K]
    cutoff = jnp.max(jnp.where(tv == thr, ti, -1), axis=1, keepdims=True)
    col = jax.lax.broadcasted_iota(jnp.int32, (ROW_BLK, S), 1)
    f_ref[...] = jnp.where(
        (zr > thr) | ((zr == thr) & (col <= cutoff)), zr, 0.0)


def _sc_decode_body(ti_hbm, tv_hbm, wdec_hbm, bdec_hbm, out_hbm,
                    idx_v, tvv_v, rows_v, acc_v, sem):
    wid = jax.lax.axis_index("s") * NC + jax.lax.axis_index("c")
    for rr in range(RPW):
        r = wid * RPW + rr
        pltpu.sync_copy(bdec_hbm, acc_v)
        pltpu.sync_copy(tv_hbm.at[r], tvv_v)
        for c in range(NCH):
            pltpu.sync_copy(ti_hbm.at[r, c], idx_v)
            pltpu.async_copy(wdec_hbm.at[idx_v], rows_v, sem).wait()
            ws = [tvv_v[c * DCH + j, :] for j in range(DCH)]

            def dbody(d, _):
                sl = pl.ds(d * 16, 16)
                av = acc_v[sl]
                for j in range(DCH):
                    av = av + ws[j] * rows_v[j, sl]
                acc_v[sl] = av
                return 0

            jax.lax.fori_loop(0, DV, dbody, 0)
        pltpu.sync_copy(acc_v, out_hbm.at[r])


_sc_decode = functools.partial(
    pl.kernel,
    mesh=plsc.VectorSubcoreMesh(core_axis_name="c", subcore_axis_name="s"),
    out_type=jax.ShapeDtypeStruct((B, D), jnp.float32),
    scratch_types=[
        pltpu.VMEM((DCH,), jnp.int32),
        pltpu.VMEM((K, 16), jnp.float32),
        pltpu.VMEM((DCH, D), jnp.float32),
        pltpu.VMEM((D,), jnp.float32),
        pltpu.SemaphoreType.DMA,
    ],
)(_sc_decode_body)


def kernel(x, W_enc, b_enc, W_dec, b_dec):
    bdec2 = b_dec.reshape(1, D)
    benc2 = b_enc.reshape(1, S)

    z = pl.pallas_call(
        _encode_block,
        grid=(S // ENC_BLK,),
        in_specs=[
            pl.BlockSpec((B, D), lambda j: (0, 0)),
            pl.BlockSpec((1, D), lambda j: (0, 0)),
            pl.BlockSpec((D, ENC_BLK), lambda j: (0, j)),
            pl.BlockSpec((1, ENC_BLK), lambda j: (0, j)),
        ],
        out_specs=pl.BlockSpec((B, ENC_BLK), lambda j: (0, j)),
        out_shape=jax.ShapeDtypeStruct((B, S), jnp.float32),
    )(x, bdec2, W_enc, benc2)

    tv, ti, f = pl.pallas_call(
        _topk_block,
        grid=(B // ROW_BLK,),
        in_specs=[pl.BlockSpec((ROW_BLK, S), lambda i: (i, 0))],
        out_specs=[
            pl.BlockSpec((ROW_BLK, K), lambda i: (i, 0)),
            pl.BlockSpec((ROW_BLK, K), lambda i: (i, 0)),
            pl.BlockSpec((ROW_BLK, S), lambda i: (i, 0)),
        ],
        out_shape=[
            jax.ShapeDtypeStruct((B, K), jnp.float32),
            jax.ShapeDtypeStruct((B, K), jnp.int32),
            jax.ShapeDtypeStruct((B, S), jnp.float32),
        ],
    )(z)

    tvx = jnp.broadcast_to(tv[:, :, None], (B, K, 16))
    x_hat = _sc_decode(ti.reshape(B, NCH, DCH), tvx, W_dec, b_dec)

    return (x_hat, f, ti, z)


# speedup vs baseline: 2.2939x; 1.0671x over previous
import functools

import jax
import jax.numpy as jnp
from jax.experimental import pallas as pl
from jax.experimental.pallas import tpu as pltpu
from jax.experimental.pallas import tpu_sc as plsc

B = 64
D = 2048
S = 65536
K = 64

ENC_BLK = 1024
ROW_BLK = 8

NC = 2
NS = 16
NW = NC * NS
RPW = B // NW
DCH = 16
NCH = K // DCH
DV = D // 16


def _encode_block(x_ref, bdec_ref, w_ref, benc_ref, z_ref):
    xc = x_ref[...] - bdec_ref[...]
    z_ref[...] = (
        jnp.dot(xc, w_ref[...], preferred_element_type=jnp.float32)
        + benc_ref[...]
    )


NSEG = 512
SEGK = 6
NCAND = NSEG * SEGK


def _topk_block(z_ref, tv_ref, ti_ref, f_ref):
    zr = jnp.maximum(z_ref[...], 0.0)
    z3 = zr.reshape(ROW_BLK, NSEG, 128)
    col3 = (jax.lax.broadcasted_iota(jnp.int32, (ROW_BLK, NSEG, 128), 1)
            * 128
            + jax.lax.broadcasted_iota(jnp.int32, (ROW_BLK, NSEG, 128), 2))

    work = z3
    cvs, cis = [], []
    for _ in range(SEGK):
        m = jnp.max(work, axis=2, keepdims=True)
        cand = jnp.where(work == m, col3, jnp.int32(S))
        am = jnp.min(cand, axis=2, keepdims=True)
        cvs.append(jnp.squeeze(m, 2))
        cis.append(jnp.squeeze(am, 2))
        work = jnp.where(cand == am, -1.0, work)

    candV = jnp.concatenate(cvs, axis=1)
    candI = jnp.concatenate(cis, axis=1)
    kcol = jax.lax.broadcasted_iota(jnp.int32, (ROW_BLK, K), 1)

    def body(k, carry):
        cv, tv, ti = carry
        m = jnp.max(cv, axis=1, keepdims=True)
        cand = jnp.where(cv == m, candI, jnp.int32(S))
        amin = jnp.min(cand, axis=1, keepdims=True)
        sel = kcol == k
        tv = jnp.where(sel, m, tv)
        ti = jnp.where(sel, amin, ti)
        cv = jnp.where(candI == amin, -1.0, cv)
        return (cv, tv, ti)

    _, tv, ti = jax.lax.fori_loop(
        0, K, body,
        (candV,
         jnp.zeros((ROW_BLK, K), jnp.float32),
         jnp.zeros((ROW_BLK, K), jnp.int32)),
    )
    tv_ref[...] = tv
    ti_ref[...] = ti
    thr = tv[:, K - 1:K]
    cutoff = jnp.max(jnp.where(tv == thr, ti, -1), axis=1, keepdims=True)
    col = jax.lax.broadcasted_iota(jnp.int32, (ROW_BLK, S), 1)
    f_ref[...] = jnp.where(
        (zr > thr) | ((zr == thr) & (col <= cutoff)), zr, 0.0)


def _sc_decode_body(ti_hbm, tv_hbm, wdec_hbm, bdec_hbm, out_hbm,
                    idx_v, tvv_v, rows_v, acc_v, sem):
    wid = jax.lax.axis_index("s") * NC + jax.lax.axis_index("c")
    for rr in range(RPW):
        r = wid * RPW + rr
        pltpu.sync_copy(bdec_hbm, acc_v)
        pltpu.sync_copy(tv_hbm.at[r], tvv_v)
        for c in range(NCH):
            pltpu.sync_copy(ti_hbm.at[r, c], idx_v)
            pltpu.async_copy(wdec_hbm.at[idx_v], rows_v, sem).wait()
            ws = [tvv_v[c * DCH + j, :] for j in range(DCH)]

            def dbody(d, _):
                sl = pl.ds(d * 16, 16)
                av = acc_v[sl]
                for j in range(DCH):
                    av = av + ws[j] * rows_v[j, sl]
                acc_v[sl] = av
                return 0

            jax.lax.fori_loop(0, DV, dbody, 0)
        pltpu.sync_copy(acc_v, out_hbm.at[r])


_sc_decode = functools.partial(
    pl.kernel,
    mesh=plsc.VectorSubcoreMesh(core_axis_name="c", subcore_axis_name="s"),
    out_type=jax.ShapeDtypeStruct((B, D), jnp.float32),
    scratch_types=[
        pltpu.VMEM((DCH,), jnp.int32),
        pltpu.VMEM((K, 16), jnp.float32),
        pltpu.VMEM((DCH, D), jnp.float32),
        pltpu.VMEM((D,), jnp.float32),
        pltpu.SemaphoreType.DMA,
    ],
)(_sc_decode_body)


def kernel(x, W_enc, b_enc, W_dec, b_dec):
    bdec2 = b_dec.reshape(1, D)
    benc2 = b_enc.reshape(1, S)

    z = pl.pallas_call(
        _encode_block,
        grid=(S // ENC_BLK,),
        in_specs=[
            pl.BlockSpec((B, D), lambda j: (0, 0)),
            pl.BlockSpec((1, D), lambda j: (0, 0)),
            pl.BlockSpec((D, ENC_BLK), lambda j: (0, j)),
            pl.BlockSpec((1, ENC_BLK), lambda j: (0, j)),
        ],
        out_specs=pl.BlockSpec((B, ENC_BLK), lambda j: (0, j)),
        out_shape=jax.ShapeDtypeStruct((B, S), jnp.float32),
    )(x, bdec2, W_enc, benc2)

    tv, ti, f = pl.pallas_call(
        _topk_block,
        grid=(B // ROW_BLK,),
        in_specs=[pl.BlockSpec((ROW_BLK, S), lambda i: (i, 0))],
        out_specs=[
            pl.BlockSpec((ROW_BLK, K), lambda i: (i, 0)),
            pl.BlockSpec((ROW_BLK, K), lambda i: (i, 0)),
            pl.BlockSpec((ROW_BLK, S), lambda i: (i, 0)),
        ],
        out_shape=[
            jax.ShapeDtypeStruct((B, K), jnp.float32),
            jax.ShapeDtypeStruct((B, K), jnp.int32),
            jax.ShapeDtypeStruct((B, S), jnp.float32),
        ],
    )(z)

    tvx = jnp.broadcast_to(tv[:, :, None], (B, K, 16))
    x_hat = _sc_decode(ti.reshape(B, NCH, DCH), tvx, W_dec, b_dec)

    return (x_hat, f, ti, z)
